# baseline (speedup 1.0000x reference)
import functools

import jax
import jax.numpy as jnp
from jax import lax
from jax.experimental import pallas as pl
from jax.experimental.pallas import tpu as pltpu
from jax.experimental.pallas import tpu_sc as plsc

N = 10000
D = 128
DH = 64
E = 320000
L = 16
NC = 2
NS = 16
CH = 128
NCHUNK = 160
NSLAB = 2
CSLAB = NCHUNK // NSLAB
E_PAD = NS * NCHUNK * CH
N_PAD = 10240
ROWS_PER_TILE = N_PAD // NS


_sc_mesh = plsc.VectorSubcoreMesh(core_axis_name="c", subcore_axis_name="s")


NGBUF = 5
NJJ = CSLAB // NGBUF


@functools.partial(
    pl.kernel,
    out_type=jax.ShapeDtypeStruct((NC, N_PAD, DH), jnp.float32),
    mesh=_sc_mesh,
    compiler_params=pltpu.CompilerParams(use_tc_tiling_on_sc=False),
    scratch_types=[
        pltpu.VMEM((CSLAB, CH), jnp.int32),
        pltpu.VMEM((CSLAB, CH), jnp.int32),
        pltpu.VMEM((CSLAB, CH), jnp.float32),
        [pltpu.VMEM((CH, DH), jnp.float32)] * NGBUF,
        [pltpu.VMEM((CH, DH), jnp.float32)] * 2,
        pltpu.VMEM_SHARED((N_PAD, DH), jnp.float32),
        [pltpu.SemaphoreType.DMA] * NGBUF,
        [pltpu.SemaphoreType.DMA] * 2,
    ],
)
def _sc_edge_agg(y_hbm, src_hbm, dst_hbm, ew_hbm, out_hbm,
                 src_v, dst_v, ew_v, gbuf, sbuf, agg_sh, sg, ss):
    c = lax.axis_index("c")
    s = lax.axis_index("s")

    def _zero_body(i, carry):
        for k in range(DH // L):
            sbuf[0][i, pl.ds(k * L, L)] = jnp.zeros((L,), jnp.float32)
        return carry

    lax.fori_loop(0, CH, _zero_body, 0)
    for r in range(ROWS_PER_TILE // CH):
        pltpu.sync_copy(
            sbuf[0], agg_sh.at[pl.ds(s * ROWS_PER_TILE + r * CH, CH)])
    plsc.subcore_barrier()

    def _gather(j, b):
        return pltpu.make_async_copy(y_hbm.at[c].at[src_v.at[j]], gbuf[b],
                                     sg[b])

    def _scatter(j, k):
        return pltpu.make_async_copy(sbuf[k], agg_sh.at[dst_v.at[j]], ss[k])

    for p in range(NSLAB):
        base = s * NCHUNK + p * CSLAB
        pltpu.sync_copy(src_hbm.at[pl.ds(base, CSLAB)], src_v)
        pltpu.sync_copy(dst_hbm.at[pl.ds(base, CSLAB)], dst_v)
        pltpu.sync_copy(ew_hbm.at[pl.ds(base, CSLAB)], ew_v)

        for b in range(NGBUF):
            _gather(b, b).start()

        def _jj_body(jj, carry):
            for b in range(NGBUF):
                j = jj * NGBUF + b
                k = b % 2
                _gather(j, b).wait()
                if b < 2:
                    @pl.when(jj > 0)
                    def _():
                        _scatter(j - 2, k).wait()
                else:
                    _scatter(j - 2, k).wait()

                @plsc.parallel_loop(0, CH // L, unroll=2)
                def _scale_group(t):
                    wvec = ew_v[j, pl.ds(t * L, L)]
                    for lane in range(L):
                        e = t * L + lane
                        w = jnp.broadcast_to(wvec[lane], (L,))
                        for kk in range(DH // L):
                            sbuf[k][e, pl.ds(kk * L, L)] = (
                                gbuf[b][e, pl.ds(kk * L, L)] * w)
                _scatter(j, k).start(add=True)

                @pl.when(jj < NJJ - 1)
                def _():
                    _gather(j + NGBUF, b).start()
            return carry

        lax.fori_loop(0, NJJ, _jj_body, 0)
        _scatter(CSLAB - 2, 0).wait()
        _scatter(CSLAB - 1, 1).wait()
    plsc.subcore_barrier()

    pltpu.sync_copy(agg_sh.at[pl.ds(s * ROWS_PER_TILE, ROWS_PER_TILE)],
                    out_hbm.at[c, pl.ds(s * ROWS_PER_TILE, ROWS_PER_TILE)])


_BLK = 1000


def _split_cols(y):
    return jnp.stack([y[:, :DH], y[:, DH:]], axis=0)


def _tc_matmul_body(x_ref, w_ref, y_ref):
    y = jnp.dot(x_ref[...], w_ref[...],
                preferred_element_type=jnp.float32)
    y_ref[...] = _split_cols(y)


def _tc_matmul(x, w):
    return pl.pallas_call(
        _tc_matmul_body,
        grid=(N // _BLK,),
        in_specs=[
            pl.BlockSpec((_BLK, D), lambda i: (i, 0)),
            pl.BlockSpec((D, D), lambda i: (0, 0)),
        ],
        out_specs=pl.BlockSpec((NC, _BLK, DH), lambda i: (0, i, 0)),
        out_shape=jax.ShapeDtypeStruct((NC, N, DH), jnp.float32),
    )(x, w)


def _tc_fuse_body(split_y, p_ref, x_ref, b_ref, g_ref, beta_ref, w_ref,
                  bn_ref, xo_ref, yo_ref):
    agg = jnp.concatenate([p_ref[0], p_ref[1]], axis=1) + b_ref[...]
    mu = jnp.mean(agg, axis=1, keepdims=True)
    dcen = agg - mu
    var = jnp.mean(dcen * dcen, axis=1, keepdims=True)
    h = dcen * lax.rsqrt(var + 1e-5) * g_ref[...] + beta_ref[...]
    h = jnp.maximum(h, 0.0)
    xn = x_ref[...] + h
    xo_ref[...] = xn
    yn = jnp.dot(xn, w_ref[...],
                 preferred_element_type=jnp.float32) + bn_ref[...]
    yo_ref[...] = _split_cols(yn) if split_y else yn


def _tc_fuse(parts, x, b, g, beta, w_next, b_next, split_y):
    if split_y:
        y_spec = pl.BlockSpec((NC, _BLK, DH), lambda i: (0, i, 0))
        y_shape = jax.ShapeDtypeStruct((NC, N, DH), jnp.float32)
    else:
        y_spec = pl.BlockSpec((_BLK, D), lambda i: (i, 0))
        y_shape = jax.ShapeDtypeStruct((N, D), jnp.float32)
    return pl.pallas_call(
        functools.partial(_tc_fuse_body, split_y),
        grid=(N // _BLK,),
        in_specs=[
            pl.BlockSpec((NC, _BLK, DH), lambda i: (0, i, 0)),
            pl.BlockSpec((_BLK, D), lambda i: (i, 0)),
            pl.BlockSpec((1, D), lambda i: (0, 0)),
            pl.BlockSpec((1, D), lambda i: (0, 0)),
            pl.BlockSpec((1, D), lambda i: (0, 0)),
            pl.BlockSpec((D, D), lambda i: (0, 0)),
            pl.BlockSpec((1, D), lambda i: (0, 0)),
        ],
        out_specs=[
            pl.BlockSpec((_BLK, D), lambda i: (i, 0)),
            y_spec,
        ],
        out_shape=[
            jax.ShapeDtypeStruct((N, D), jnp.float32),
            y_shape,
        ],
    )(parts, x, b, g, beta, w_next, b_next)


def kernel(x, edge_index, edge_weight, Ws, bs, gs, betas, Wp, bp):
    src = jnp.asarray(edge_index[0], jnp.int32)
    dst = jnp.asarray(edge_index[1], jnp.int32)
    ew = jnp.asarray(edge_weight, jnp.float32)

    pad = E_PAD - E
    pad_idx = jnp.arange(pad, dtype=jnp.int32) % N
    src_p = jnp.concatenate([src, pad_idx]).reshape(E_PAD // CH, CH)
    dst_p = jnp.concatenate([dst, pad_idx]).reshape(E_PAD // CH, CH)
    ew_p = jnp.concatenate([ew, jnp.zeros((pad,), jnp.float32)]
                           ).reshape(E_PAD // CH, CH)

    zrow = jnp.zeros((1, D), jnp.float32)
    y = _tc_matmul(x, Ws[0])
    for i in range(4):
        parts = _sc_edge_agg(y, src_p, dst_p, ew_p)
        last = i == 3
        w_next = Wp if last else Ws[i + 1]
        b_next = bp.reshape(1, D) if last else zrow
        x, y = _tc_fuse(parts, x, bs[i].reshape(1, D), gs[i].reshape(1, D),
                        betas[i].reshape(1, D), w_next, b_next,
                        split_y=not last)
    return y


# speedup vs baseline: 1.0299x; 1.0299x over previous
import functools

import jax
import jax.numpy as jnp
from jax import lax
from jax.experimental import pallas as pl
from jax.experimental.pallas import tpu as pltpu
from jax.experimental.pallas import tpu_sc as plsc

N = 10000
D = 128
DH = 64
E = 320000
L = 16
NC = 2
NS = 16
CH = 128
NCHUNK = 160
NSLAB = 2
CSLAB = NCHUNK // NSLAB
E_PAD = NS * NCHUNK * CH
N_PAD = 10240
ZROWS = 64
ROWS_PER_TILE = N_PAD // NS


_sc_mesh = plsc.VectorSubcoreMesh(core_axis_name="c", subcore_axis_name="s")


NGBUF = 4
NJJ = CSLAB // NGBUF


@functools.partial(
    pl.kernel,
    out_type=jax.ShapeDtypeStruct((NC, N_PAD, DH), jnp.float32),
    mesh=_sc_mesh,
    compiler_params=pltpu.CompilerParams(use_tc_tiling_on_sc=False),
    scratch_types=[
        pltpu.VMEM((CSLAB, CH), jnp.int32),
        pltpu.VMEM((CSLAB, CH), jnp.int32),
        pltpu.VMEM((CSLAB, CH), jnp.float32),
        [pltpu.VMEM((CH, DH), jnp.float32)] * NGBUF,
        [pltpu.VMEM((CH, DH), jnp.float32)] * 2,
        pltpu.VMEM((ZROWS, DH), jnp.float32),
        pltpu.VMEM_SHARED((N_PAD, DH), jnp.float32),
        [pltpu.SemaphoreType.DMA] * NGBUF,
        [pltpu.SemaphoreType.DMA] * 2,
    ],
)
def _sc_edge_agg(y_hbm, src_hbm, dst_hbm, ew_hbm, out_hbm,
                 src_v, dst_v, ew_v, gbuf, sbuf, zero_v, agg_sh, sg, ss):
    c = lax.axis_index("c")
    s = lax.axis_index("s")

    def _zero_body(i, carry):
        for k in range(DH // L):
            zero_v[i, pl.ds(k * L, L)] = jnp.zeros((L,), jnp.float32)
        return carry

    lax.fori_loop(0, ZROWS, _zero_body, 0)
    for r in range(ROWS_PER_TILE // ZROWS):
        pltpu.sync_copy(
            zero_v, agg_sh.at[pl.ds(s * ROWS_PER_TILE + r * ZROWS, ZROWS)])
    plsc.subcore_barrier()

    def _gather(j, b):
        return pltpu.make_async_copy(y_hbm.at[c].at[src_v.at[j]], gbuf[b],
                                     sg[b])

    def _scatter(j, k):
        return pltpu.make_async_copy(sbuf[k], agg_sh.at[dst_v.at[j]], ss[k])

    for p in range(NSLAB):
        base = s * NCHUNK + p * CSLAB
        pltpu.sync_copy(src_hbm.at[pl.ds(base, CSLAB)], src_v)
        pltpu.sync_copy(dst_hbm.at[pl.ds(base, CSLAB)], dst_v)
        pltpu.sync_copy(ew_hbm.at[pl.ds(base, CSLAB)], ew_v)

        for b in range(NGBUF):
            _gather(b, b).start()

        def _jj_body(jj, carry):
            for b in range(NGBUF):
                j = jj * NGBUF + b
                k = b % 2
                _gather(j, b).wait()
                if b < 2:
                    @pl.when(jj > 0)
                    def _():
                        _scatter(j - 2, k).wait()
                else:
                    _scatter(j - 2, k).wait()

                @plsc.parallel_loop(0, CH // L, unroll=2)
                def _scale_group(t):
                    wvec = ew_v[j, pl.ds(t * L, L)]
                    for lane in range(L):
                        e = t * L + lane
                        w = jnp.broadcast_to(wvec[lane], (L,))
                        for kk in range(DH // L):
                            sbuf[k][e, pl.ds(kk * L, L)] = (
                                gbuf[b][e, pl.ds(kk * L, L)] * w)
                _scatter(j, k).start(add=True)

                @pl.when(jj < NJJ - 1)
                def _():
                    _gather(j + NGBUF, b).start()
            return carry

        lax.fori_loop(0, NJJ, _jj_body, 0)
        _scatter(CSLAB - 2, 0).wait()
        _scatter(CSLAB - 1, 1).wait()
    plsc.subcore_barrier()

    pltpu.sync_copy(agg_sh.at[pl.ds(s * ROWS_PER_TILE, ROWS_PER_TILE)],
                    out_hbm.at[c, pl.ds(s * ROWS_PER_TILE, ROWS_PER_TILE)])


_BLK = 2000


def _split_cols(y):
    return jnp.stack([y[:, :DH], y[:, DH:]], axis=0)


def _tc_matmul_body(x_ref, w_ref, y_ref):
    y = jnp.dot(x_ref[...], w_ref[...],
                preferred_element_type=jnp.float32)
    y_ref[...] = _split_cols(y)


def _tc_matmul(x, w):
    return pl.pallas_call(
        _tc_matmul_body,
        grid=(N // _BLK,),
        in_specs=[
            pl.BlockSpec((_BLK, D), lambda i: (i, 0)),
            pl.BlockSpec((D, D), lambda i: (0, 0)),
        ],
        out_specs=pl.BlockSpec((NC, _BLK, DH), lambda i: (0, i, 0)),
        out_shape=jax.ShapeDtypeStruct((NC, N, DH), jnp.float32),
    )(x, w)


def _tc_fuse_body(split_y, p_ref, x_ref, b_ref, g_ref, beta_ref, w_ref,
                  bn_ref, xo_ref, yo_ref):
    agg = jnp.concatenate([p_ref[0], p_ref[1]], axis=1) + b_ref[...]
    mu = jnp.mean(agg, axis=1, keepdims=True)
    dcen = agg - mu
    var = jnp.mean(dcen * dcen, axis=1, keepdims=True)
    h = dcen * lax.rsqrt(var + 1e-5) * g_ref[...] + beta_ref[...]
    h = jnp.maximum(h, 0.0)
    xn = x_ref[...] + h
    xo_ref[...] = xn
    yn = jnp.dot(xn, w_ref[...],
                 preferred_element_type=jnp.float32) + bn_ref[...]
    yo_ref[...] = _split_cols(yn) if split_y else yn


def _tc_fuse(parts, x, b, g, beta, w_next, b_next, split_y):
    if split_y:
        y_spec = pl.BlockSpec((NC, _BLK, DH), lambda i: (0, i, 0))
        y_shape = jax.ShapeDtypeStruct((NC, N, DH), jnp.float32)
    else:
        y_spec = pl.BlockSpec((_BLK, D), lambda i: (i, 0))
        y_shape = jax.ShapeDtypeStruct((N, D), jnp.float32)
    return pl.pallas_call(
        functools.partial(_tc_fuse_body, split_y),
        grid=(N // _BLK,),
        in_specs=[
            pl.BlockSpec((NC, _BLK, DH), lambda i: (0, i, 0)),
            pl.BlockSpec((_BLK, D), lambda i: (i, 0)),
            pl.BlockSpec((1, D), lambda i: (0, 0)),
            pl.BlockSpec((1, D), lambda i: (0, 0)),
            pl.BlockSpec((1, D), lambda i: (0, 0)),
            pl.BlockSpec((D, D), lambda i: (0, 0)),
            pl.BlockSpec((1, D), lambda i: (0, 0)),
        ],
        out_specs=[
            pl.BlockSpec((_BLK, D), lambda i: (i, 0)),
            y_spec,
        ],
        out_shape=[
            jax.ShapeDtypeStruct((N, D), jnp.float32),
            y_shape,
        ],
    )(parts, x, b, g, beta, w_next, b_next)


def kernel(x, edge_index, edge_weight, Ws, bs, gs, betas, Wp, bp):
    src = jnp.asarray(edge_index[0], jnp.int32)
    dst = jnp.asarray(edge_index[1], jnp.int32)
    ew = jnp.asarray(edge_weight, jnp.float32)

    pad = E_PAD - E
    pad_idx = jnp.arange(pad, dtype=jnp.int32) % N
    src_p = jnp.concatenate([src, pad_idx]).reshape(E_PAD // CH, CH)
    dst_p = jnp.concatenate([dst, pad_idx]).reshape(E_PAD // CH, CH)
    ew_p = jnp.concatenate([ew, jnp.zeros((pad,), jnp.float32)]
                           ).reshape(E_PAD // CH, CH)

    zrow = jnp.zeros((1, D), jnp.float32)
    y = _tc_matmul(x, Ws[0])
    for i in range(4):
        parts = _sc_edge_agg(y, src_p, dst_p, ew_p)
        last = i == 3
        w_next = Wp if last else Ws[i + 1]
        b_next = bp.reshape(1, D) if last else zrow
        x, y = _tc_fuse(parts, x, bs[i].reshape(1, D), gs[i].reshape(1, D),
                        betas[i].reshape(1, D), w_next, b_next,
                        split_y=not last)
    return y
